# baseline (speedup 1.0000x reference)
import functools

import jax
import jax.numpy as jnp
from jax.experimental import pallas as pl
from jax.experimental.pallas import tpu as pltpu

_SOFT = 0.1
_HI = jax.lax.Precision.HIGHEST
_TI = 1024


def _dot(a, b, dims):
    return jax.lax.dot_general(a, b, (dims, ((), ())), precision=_HI,
                               preferred_element_type=jnp.float32)


def _step_kernel(sdt_ref, pj_ref, pi_ref, vi_ref, w1_ref, b1_ref, w2_ref,
                 b2_ref, *out_refs, update_pos):
    pj = pj_ref[0]
    pi = pi_ref[0]
    vi = vi_ref[0]
    sdt = sdt_ref[0, 0]

    h = jnp.tanh(_dot(pj, w1_ref[...], ((1,), (0,))) + b1_ref[...])
    m = jax.nn.softplus(_dot(h, w2_ref[...], ((1,), (0,))) + b2_ref[...])

    nj = jnp.sum(pj * pj, axis=1, keepdims=True)
    ni = jnp.sum(pi * pi, axis=1, keepdims=True)
    pj_aug = jnp.concatenate([pj, jnp.ones_like(nj), nj], axis=1)
    pi_aug = jnp.concatenate([-2.0 * pi, ni, jnp.ones_like(ni)], axis=1)
    dist2 = _dot(pi_aug, pj_aug, ((1,), (1,))) + _SOFT
    r = jax.lax.rsqrt(dist2)
    inv = r * r * r

    q = jnp.concatenate([m * pj, m], axis=1)
    a = _dot(inv, q, ((1,), (0,)))
    acc = a[:, 0:3] - a[:, 3:4] * pi

    v_new = vi + 0.5 * sdt * acc
    if update_pos:
        out_refs[0][0] = pi + sdt * v_new
        out_refs[1][0] = v_new
    else:
        out_refs[0][0] = v_new


def _half_kick(p, v, sdt, W1, b1, W2, b2, update_pos):
    B, N, D = p.shape
    H = W1.shape[1]
    grid = (B, N // _TI)
    spec_full = pl.BlockSpec((1, N, D), lambda b, i: (b, 0, 0))
    spec_tile = pl.BlockSpec((1, _TI, D), lambda b, i: (b, i, 0))

    def whole(shp):
        return pl.BlockSpec(shp, lambda b, i: (0,) * len(shp))

    in_specs = [
        pl.BlockSpec(memory_space=pltpu.SMEM),
        spec_full, spec_tile, spec_tile,
        whole((D, H)), whole((1, H)), whole((H, 1)), whole((1, 1)),
    ]
    n_out = 2 if update_pos else 1
    out = pl.pallas_call(
        functools.partial(_step_kernel, update_pos=update_pos),
        grid=grid,
        in_specs=in_specs,
        out_specs=[spec_tile] * n_out,
        out_shape=[jax.ShapeDtypeStruct((B, N, D), jnp.float32)] * n_out,
    )(sdt, p, p, v, W1, b1, W2, b2)
    return out


def kernel(pos, vel, t_span, dt, W1, b1, W2, b2):
    p0 = pos.astype(jnp.float32)
    v0 = vel.astype(jnp.float32)
    dtf = jnp.asarray(dt, dtype=jnp.float32)
    step_dt = jnp.minimum(dtf, t_span[1].astype(jnp.float32))
    sdt = jnp.reshape(step_dt, (1, 1))
    b1r = jnp.reshape(b1, (1, -1)).astype(jnp.float32)
    b2r = jnp.reshape(b2, (1, 1)).astype(jnp.float32)
    W1f = W1.astype(jnp.float32)
    W2f = W2.astype(jnp.float32)

    p1, v_half = _half_kick(p0, v0, sdt, W1f, b1r, W2f, b2r, update_pos=True)
    (v1,) = _half_kick(p1, v_half, sdt, W1f, b1r, W2f, b2r, update_pos=False)

    snap0 = jnp.concatenate([p0, v0], axis=-1)
    snap1 = jnp.concatenate([p1, v1], axis=-1)
    return jnp.stack([snap0, snap1], axis=1)


# speedup vs baseline: 2.2039x; 2.2039x over previous
import functools

import jax
import jax.numpy as jnp
from jax.experimental import pallas as pl
from jax.experimental.pallas import tpu as pltpu

_SOFT = 0.1
_TI = 1024


def _dot1(a, b, dims):
    return jax.lax.dot_general(a, b, (dims, ((), ())),
                               precision=jax.lax.Precision.DEFAULT,
                               preferred_element_type=jnp.float32)


def _dot3(a, b, dims):
    ah = a.astype(jnp.bfloat16).astype(jnp.float32)
    bh = b.astype(jnp.bfloat16).astype(jnp.float32)
    return (_dot1(a, b, dims) + _dot1(a - ah, b, dims)
            + _dot1(a, b - bh, dims))


def _step_kernel(sdt_ref, pj_ref, pi_ref, vi_ref, w1_ref, b1_ref, w2_ref,
                 b2_ref, *out_refs, update_pos):
    pj = pj_ref[0]
    pi = pi_ref[0]
    vi = vi_ref[0]
    sdt = sdt_ref[0, 0]

    h = jnp.tanh(_dot3(pj, w1_ref[...], ((1,), (0,))) + b1_ref[...])
    m = jax.nn.softplus(_dot3(h, w2_ref[...], ((1,), (0,))) + b2_ref[...])

    nj = jnp.sum(pj * pj, axis=1, keepdims=True)
    ni = jnp.sum(pi * pi, axis=1, keepdims=True)
    pj_aug = jnp.concatenate([pj, jnp.ones_like(nj), nj], axis=1)
    pi_aug = jnp.concatenate([-2.0 * pi, ni, jnp.ones_like(ni)], axis=1)
    dist2 = _dot3(pi_aug, pj_aug, ((1,), (1,))) + _SOFT
    r = jax.lax.rsqrt(dist2)
    inv = r * r * r

    q = jnp.concatenate([m * pj, m], axis=1)
    a = _dot1(inv, q, ((1,), (0,)))
    acc = a[:, 0:3] - a[:, 3:4] * pi

    v_new = vi + 0.5 * sdt * acc
    if update_pos:
        out_refs[0][0] = pi + sdt * v_new
        out_refs[1][0] = v_new
    else:
        out_refs[0][0] = v_new


def _half_kick(p, v, sdt, W1, b1, W2, b2, update_pos):
    B, N, D = p.shape
    H = W1.shape[1]
    grid = (B, N // _TI)
    spec_full = pl.BlockSpec((1, N, D), lambda b, i: (b, 0, 0))
    spec_tile = pl.BlockSpec((1, _TI, D), lambda b, i: (b, i, 0))

    def whole(shp):
        return pl.BlockSpec(shp, lambda b, i: (0,) * len(shp))

    in_specs = [
        pl.BlockSpec(memory_space=pltpu.SMEM),
        spec_full, spec_tile, spec_tile,
        whole((D, H)), whole((1, H)), whole((H, 1)), whole((1, 1)),
    ]
    n_out = 2 if update_pos else 1
    out = pl.pallas_call(
        functools.partial(_step_kernel, update_pos=update_pos),
        grid=grid,
        in_specs=in_specs,
        out_specs=[spec_tile] * n_out,
        out_shape=[jax.ShapeDtypeStruct((B, N, D), jnp.float32)] * n_out,
    )(sdt, p, p, v, W1, b1, W2, b2)
    return out


def kernel(pos, vel, t_span, dt, W1, b1, W2, b2):
    p0 = pos.astype(jnp.float32)
    v0 = vel.astype(jnp.float32)
    dtf = jnp.asarray(dt, dtype=jnp.float32)
    step_dt = jnp.minimum(dtf, t_span[1].astype(jnp.float32))
    sdt = jnp.reshape(step_dt, (1, 1))
    b1r = jnp.reshape(b1, (1, -1)).astype(jnp.float32)
    b2r = jnp.reshape(b2, (1, 1)).astype(jnp.float32)
    W1f = W1.astype(jnp.float32)
    W2f = W2.astype(jnp.float32)

    p1, v_half = _half_kick(p0, v0, sdt, W1f, b1r, W2f, b2r, update_pos=True)
    (v1,) = _half_kick(p1, v_half, sdt, W1f, b1r, W2f, b2r, update_pos=False)

    snap0 = jnp.concatenate([p0, v0], axis=-1)
    snap1 = jnp.concatenate([p1, v1], axis=-1)
    return jnp.stack([snap0, snap1], axis=1)
